# Initial kernel scaffold; baseline (speedup 1.0000x reference)
import jax
import jax.numpy as jnp
from jax.experimental import pallas as pl


def kernel(x, mask_add, slab):
    raise NotImplementedError("write your pallas kernel here")


# speedup vs baseline: 8.2226x; 8.2226x over previous
import jax
import jax.numpy as jnp
from jax.experimental import pallas as pl
from jax.experimental.pallas import tpu as pltpu

N = 16
IN_FEATS = 16
HIDDEN = 8
HEADS = 4
NEG_SLOPE = 0.2
NEG_INF = -1e9

_W1 = (0, 16, 32)
_ATTN1 = (16, 32, 8)
_B1 = (48, 1, 32)
_W2 = (192, 32, 8)
_ATTN2 = (224, 8, 2)
_B2 = (232, 1, 8)
_MW1 = (264, 8, 16)
_MB1 = (272, 1, 16)
_MW2 = (280, 16, 4)
_MB2 = (296, 1, 4)


def _leaky_relu(x):
    return jnp.where(x > 0, x, NEG_SLOPE * x)


def _make_body(chunks):
    def _w(w_ref, spec):
        off, r, c = spec
        return w_ref[off:off + r, 0:c]

    def body(x_ref, mask_ref, w_ref, node_ref, graph_ref):
        f32 = jnp.float32
        w1 = _w(w_ref, _W1)
        attn1 = _w(w_ref, _ATTN1)
        b1 = _w(w_ref, _B1)
        w2 = _w(w_ref, _W2)
        attn2 = _w(w_ref, _ATTN2)
        b2 = _w(w_ref, _B2)
        mw1 = _w(w_ref, _MW1)
        mb1 = _w(w_ref, _MB1)
        mw2 = _w(w_ref, _MW2)
        mb2 = _w(w_ref, _MB2)

        rows = jax.lax.broadcasted_iota(jnp.int32, (128, 128), 0)
        cols = jax.lax.broadcasted_iota(jnp.int32, (128, 128), 1)
        blk = (rows >> 4) == (cols >> 4)
        u16 = jax.lax.broadcasted_iota(jnp.int32, (16, 128), 0)
        j16 = jax.lax.broadcasted_iota(jnp.int32, (16, 128), 1)
        tile16 = ((j16 & 15) == u16).astype(f32)
        g8 = jax.lax.broadcasted_iota(jnp.int32, (8, 128), 0)
        j8 = jax.lax.broadcasted_iota(jnp.int32, (8, 128), 1)
        rsel = jnp.where((j8 >> 4) == g8, 1.0 / N, 0.0).astype(f32)

        xall = x_ref[...].reshape(chunks * 128, IN_FEATS)
        mall = mask_ref[...].reshape(chunks * 128, N)

        for c in range(chunks):
            x = xall[c * 128:(c + 1) * 128]
            mask = mall[c * 128:(c + 1) * 128]

            mask_t = jnp.dot(mask, tile16, preferred_element_type=f32)

            feat1 = jnp.dot(x, w1, preferred_element_type=f32)
            elr1 = jnp.dot(feat1, attn1, preferred_element_type=f32)
            elr1_t = jnp.transpose(elr1)

            aggs = []
            for h in range(HEADS):
                er_col = elr1[:, HEADS + h:HEADS + h + 1]
                el_row = elr1_t[h:h + 1, :]
                e = _leaky_relu(er_col + el_row)
                e = jnp.where(blk, e + mask_t, NEG_INF)
                m = jnp.max(e, axis=1, keepdims=True)
                p = jnp.exp(e - m)
                s = jnp.sum(p, axis=1, keepdims=True)
                agg = jnp.dot(p, feat1[:, h * HIDDEN:(h + 1) * HIDDEN],
                              preferred_element_type=f32)
                aggs.append(agg * pl.reciprocal(s, approx=True))
            h1 = jnp.concatenate(aggs, axis=1) + b1

            feat2 = jnp.dot(h1, w2, preferred_element_type=f32)
            elr2 = jnp.dot(feat2, attn2, preferred_element_type=f32)
            el2 = jnp.transpose(elr2)[0:1, :]
            er2 = elr2[:, 1:2]
            e2 = _leaky_relu(er2 + el2)
            e2 = jnp.where(blk, e2 + mask_t, NEG_INF)
            m2 = jnp.max(e2, axis=1, keepdims=True)
            p2 = jnp.exp(e2 - m2)
            s2 = jnp.sum(p2, axis=1, keepdims=True)
            h2 = jnp.dot(p2, feat2, preferred_element_type=f32)
            h2 = h2 * pl.reciprocal(s2, approx=True) + b2

            hg = jnp.dot(rsel, h2, preferred_element_type=f32)
            hc = jnp.concatenate([h2, hg], axis=0)
            hid = jnp.maximum(jnp.dot(hc, mw1, preferred_element_type=f32) + mb1, 0.0)
            logits = jnp.dot(hid, mw2, preferred_element_type=f32) + mb2

            node_ref[c * 8:(c + 1) * 8] = logits[0:128, 0:2].reshape(8, N, 2)
            graph_ref[c * 8:(c + 1) * 8, :] = logits[128:136, 2:4]

    return body


def kernel(x, mask_add, slab):
    b = x.shape[0]
    chunks = 4 if b % 32 == 0 else 1
    g = 8 * chunks
    node_logits, graph_logits = pl.pallas_call(
        _make_body(chunks),
        out_shape=(
            jax.ShapeDtypeStruct((b, N, 2), jnp.float32),
            jax.ShapeDtypeStruct((b, 2), jnp.float32),
        ),
        grid=(b // g,),
        in_specs=[
            pl.BlockSpec((g, N, IN_FEATS), lambda i: (i, 0, 0)),
            pl.BlockSpec((g, N, N), lambda i: (i, 0, 0)),
            pl.BlockSpec(slab.shape, lambda i: (0, 0)),
        ],
        out_specs=(
            pl.BlockSpec((g, N, 2), lambda i: (i, 0, 0)),
            pl.BlockSpec((g, 2), lambda i: (i, 0)),
        ),
        compiler_params=pltpu.CompilerParams(
            dimension_semantics=("parallel",),
        ),
    )(x, mask_add, slab)
    return node_logits, graph_logits


# speedup vs baseline: 15.0506x; 1.8304x over previous
import jax
import jax.numpy as jnp
from jax.experimental import pallas as pl
from jax.experimental.pallas import tpu as pltpu

N = 16
IN_FEATS = 16
HIDDEN = 8
HEADS = 4
NEG_SLOPE = 0.2
NEG_INF = -1e9
CHUNKS = 4

_W1 = (0, 16, 32)
_ATTN1 = (16, 32, 8)
_B1 = (48, 1, 32)
_W2 = (192, 32, 8)
_ATTN2 = (224, 8, 2)
_B2 = (232, 1, 8)
_MW1 = (264, 8, 16)
_MB1 = (272, 1, 16)
_MW2 = (280, 16, 4)
_MB2 = (296, 1, 4)


def _leaky_relu(x):
    return jnp.maximum(x, NEG_SLOPE * x)


def _make_body(chunks):
    rows = chunks * 128

    def _w(w_ref, spec):
        off, r, c = spec
        return w_ref[off:off + r, 0:c]

    def body(x_ref, mask_ref, w_ref, node_ref, graph_ref):
        f32 = jnp.float32
        w1 = _w(w_ref, _W1)
        attn1 = _w(w_ref, _ATTN1)
        b1 = _w(w_ref, _B1)
        w2 = _w(w_ref, _W2)
        attn2 = _w(w_ref, _ATTN2)
        b2 = _w(w_ref, _B2)
        mw1 = _w(w_ref, _MW1)
        mb1 = _w(w_ref, _MB1)
        mw2 = _w(w_ref, _MW2)
        mb2 = _w(w_ref, _MB2)

        u16 = jax.lax.broadcasted_iota(jnp.int32, (16, 128), 0)
        j16 = jax.lax.broadcasted_iota(jnp.int32, (16, 128), 1)
        tile16 = ((j16 & 15) == u16).astype(f32)
        r2 = jax.lax.broadcasted_iota(jnp.int32, (rows, 128), 0)
        c2 = jax.lax.broadcasted_iota(jnp.int32, (rows, 128), 1)
        offmask = jnp.where(((r2 >> 4) & 7) == (c2 >> 4), 0.0, NEG_INF).astype(f32)
        rg = jax.lax.broadcasted_iota(jnp.int32, (chunks * 8, rows), 0)
        jg = jax.lax.broadcasted_iota(jnp.int32, (chunks * 8, rows), 1)
        rsel = jnp.where((jg >> 4) == rg, 1.0 / N, 0.0).astype(f32)
        ones_col = jnp.full((128, 1), 1.0, f32)

        xall = x_ref[...].reshape(rows, IN_FEATS)
        mall = mask_ref[...].reshape(rows, N)

        maskc = jnp.dot(mall, tile16, preferred_element_type=f32) + offmask

        feat1 = jnp.dot(xall, w1, preferred_element_type=f32)
        elr1 = jnp.dot(feat1, attn1, preferred_element_type=f32)
        elr1_t = jnp.transpose(elr1)

        h1_parts = []
        for h in range(HEADS):
            er_col = elr1[:, HEADS + h:HEADS + h + 1]
            el_tiled = jnp.concatenate(
                [jnp.broadcast_to(elr1_t[h:h + 1, c * 128:(c + 1) * 128], (128, 128))
                 for c in range(chunks)], axis=0)
            e = _leaky_relu(er_col + el_tiled) + maskc
            m = jnp.max(e, axis=1, keepdims=True)
            p = jnp.exp(e - m)
            s = jnp.dot(p, ones_col, preferred_element_type=f32)
            agg = jnp.concatenate(
                [jnp.dot(p[c * 128:(c + 1) * 128],
                         feat1[c * 128:(c + 1) * 128, h * HIDDEN:(h + 1) * HIDDEN],
                         preferred_element_type=f32)
                 for c in range(chunks)], axis=0)
            h1_parts.append(agg * pl.reciprocal(s, approx=True))
        h1 = jnp.concatenate(h1_parts, axis=1) + b1

        feat2 = jnp.dot(h1, w2, preferred_element_type=f32)
        elr2 = jnp.dot(feat2, attn2, preferred_element_type=f32)
        elr2_t = jnp.transpose(elr2)
        el2_tiled = jnp.concatenate(
            [jnp.broadcast_to(elr2_t[0:1, c * 128:(c + 1) * 128], (128, 128))
             for c in range(chunks)], axis=0)
        er2 = elr2[:, 1:2]
        e2 = _leaky_relu(er2 + el2_tiled) + maskc
        m2 = jnp.max(e2, axis=1, keepdims=True)
        p2 = jnp.exp(e2 - m2)
        s2 = jnp.dot(p2, ones_col, preferred_element_type=f32)
        h2 = jnp.concatenate(
            [jnp.dot(p2[c * 128:(c + 1) * 128], feat2[c * 128:(c + 1) * 128],
                     preferred_element_type=f32)
             for c in range(chunks)], axis=0)
        h2 = h2 * pl.reciprocal(s2, approx=True) + b2

        hg = jnp.dot(rsel, h2, preferred_element_type=f32)
        hc = jnp.concatenate([h2, hg], axis=0)
        hid = jnp.maximum(jnp.dot(hc, mw1, preferred_element_type=f32) + mb1, 0.0)
        logits = jnp.dot(hid, mw2, preferred_element_type=f32) + mb2

        node_ref[...] = logits[0:rows, 0:2].reshape(chunks * 8, N, 2)
        graph_ref[...] = logits[rows:rows + chunks * 8, 2:4]

    return body


def kernel(x, mask_add, slab):
    b = x.shape[0]
    chunks = CHUNKS if b % (8 * CHUNKS) == 0 else 1
    g = 8 * chunks
    node_logits, graph_logits = pl.pallas_call(
        _make_body(chunks),
        out_shape=(
            jax.ShapeDtypeStruct((b, N, 2), jnp.float32),
            jax.ShapeDtypeStruct((b, 2), jnp.float32),
        ),
        grid=(b // g,),
        in_specs=[
            pl.BlockSpec((g, N, IN_FEATS), lambda i: (i, 0, 0)),
            pl.BlockSpec((g, N, N), lambda i: (i, 0, 0)),
            pl.BlockSpec(slab.shape, lambda i: (0, 0)),
        ],
        out_specs=(
            pl.BlockSpec((g, N, 2), lambda i: (i, 0, 0)),
            pl.BlockSpec((g, 2), lambda i: (i, 0)),
        ),
        compiler_params=pltpu.CompilerParams(
            dimension_semantics=("parallel",),
        ),
    )(x, mask_add, slab)
    return node_logits, graph_logits
